# baseline (speedup 1.0000x reference)
import functools

import jax
import jax.numpy as jnp
from jax import lax
from jax.experimental import pallas as pl
from jax.experimental.pallas import tpu as pltpu
from jax.experimental.pallas import tpu_sc as plsc

_N = 10000
_NSC = 10240
_E_LAP = 163840
_E_BND = 40960
_E_PAD = 4 * _E_LAP + 6 * _E_BND
_C = 512

_BLOCKS = (
    (0, 0, ('lap', 0), 0),
    (1, 0, ('lap', 1), 1),
    (2, 0, ('lap', 2), 2),
    (3, 0, ('lap', 3), 3),
    (1, 1, ('bndT', 1), 0),
    (2, 1, ('bndT', 2), 1),
    (3, 1, ('bndT', 3), 2),
    (0, 2, ('bnd', 1), 1),
    (1, 2, ('bnd', 2), 2),
    (2, 2, ('bnd', 3), 3),
)
_STARTS = []
_off = 0
for _b in _BLOCKS:
    _STARTS.append(_off)
    _off += _E_LAP if _b[2][0] == 'lap' else _E_BND
_SO = tuple(b[3] for b in _BLOCKS)


_WH = 64


def _make_sc_spmm(npass):
    mesh = plsc.VectorSubcoreMesh(core_axis_name="c", subcore_axis_name="s")
    wh = _WH

    @functools.partial(
        pl.kernel, mesh=mesh,
        out_type=jax.ShapeDtypeStruct((2 * npass * 10 * _NSC, wh),
                                      jnp.float32),
        scratch_types=[
            [pltpu.VMEM((128,), jnp.int32)] * 4,
            [pltpu.VMEM((128,), jnp.int32)] * 4,
            pltpu.VMEM((_C,), jnp.float32),
            pltpu.VMEM((_C, wh), jnp.float32),
            pltpu.VMEM((128, wh), jnp.float32),
            pltpu.VMEM_SHARED((_NSC, wh), jnp.float32),
            pltpu.SemaphoreType.DMA,
        ],
        compiler_params=pltpu.CompilerParams(use_tc_tiling_on_sc=False),
    )
    def spmm_kernel(x_hbm, src_hbm, dst_hbm, val_hbm, z_hbm,
                    src_s, dst_s, val_s, rows, stage, acc, sem):
        c = lax.axis_index("c")
        s = lax.axis_index("s")

        def zero_stage(_r, carry):
            for j in range(wh // 16):
                stage[_r, pl.ds(j * 16, 16)] = jnp.zeros((16,), jnp.float32)
            return carry

        def scale_group(g, carry):
            val16 = val_s[pl.ds(g * 16, 16)]
            for L in range(16):
                vv = jnp.broadcast_to(lax.slice(val16, (L,), (L + 1,)), (16,))
                r = g * 16 + L
                for j in range(wh // 16):
                    rows[r, pl.ds(j * 16, 16)] = (
                        rows[r, pl.ds(j * 16, 16)] * vv)
            return carry

        def run_block(start, ntile, cofs, zbase):
            lax.fori_loop(0, 128, zero_stage, 0)
            for t in range(5):
                pltpu.sync_copy(stage, acc.at[pl.ds(s * 640 + t * 128, 128)])
            plsc.subcore_barrier()

            def chunk(kk, carry):
                off = start + (kk * 16 + s) * _C
                for q in range(4):
                    pltpu.sync_copy(src_hbm.at[pl.ds(off + q * 128, 128)],
                                    src_s[q])
                    pltpu.sync_copy(dst_hbm.at[pl.ds(off + q * 128, 128)],
                                    dst_s[q])
                pltpu.sync_copy(val_hbm.at[pl.ds(off, _C)], val_s)
                for q in range(4):
                    for j in range(8):
                        sl = src_s[q][pl.ds(j * 16, 16)]
                        src_s[q][pl.ds(j * 16, 16)] = sl + cofs
                for q in range(4):
                    pltpu.async_copy(x_hbm.at[src_s[q]],
                                     rows.at[pl.ds(q * 128, 128)], sem).wait()
                lax.fori_loop(0, _C // 16, scale_group, 0)
                for q in range(4):
                    pltpu.sync_copy(rows.at[pl.ds(q * 128, 128)],
                                    acc.at[dst_s[q]], add=True)
                return carry

            lax.fori_loop(0, ntile, chunk, 0)
            plsc.subcore_barrier()
            for t in range(5):
                r0 = s * 640 + t * 128
                pltpu.sync_copy(acc.at[pl.ds(r0, 128)], stage)
                pltpu.sync_copy(stage, z_hbm.at[pl.ds(zbase + r0, 128)])
            plsc.subcore_barrier()

        for h in range(npass):
            q = c * npass + h
            cofs = q * (4 * _N)

            def lap_blk(b, carry):
                run_block(b * _E_LAP, _E_LAP // (16 * _C), cofs,
                          (q * 10 + b) * _NSC)
                return carry

            def bnd_blk(i, carry):
                run_block(4 * _E_LAP + i * _E_BND, _E_BND // (16 * _C), cofs,
                          (q * 10 + 4 + i) * _NSC)
                return carry

            lax.fori_loop(0, 4, lap_blk, 0)
            lax.fori_loop(0, 6, bnd_blk, 0)

    return spmm_kernel


def _make_sc_wvec():
    mesh = plsc.VectorSubcoreMesh(core_axis_name="c", subcore_axis_name="s")

    @functools.partial(
        pl.kernel, mesh=mesh,
        out_type=jax.ShapeDtypeStruct((2 * 10 * _NSC, 16), jnp.float32),
        scratch_types=[
            [pltpu.VMEM((128,), jnp.int32)] * 4,
            pltpu.VMEM((_C,), jnp.int32),
            pltpu.VMEM((_C,), jnp.float32),
            pltpu.VMEM((_C, 16), jnp.float32),
            pltpu.VMEM((128, 16), jnp.float32),
            pltpu.VMEM((640, 16), jnp.float32),
            pltpu.VMEM((16,), jnp.int32),
            pltpu.VMEM_SHARED((_NSC, 16), jnp.float32),
        ],
        compiler_params=pltpu.CompilerParams(use_tc_tiling_on_sc=False),
    )
    def wvec_kernel(src_hbm, dst_hbm, val_hbm, idx_hbm, w_hbm,
                    srcw_s, dst_s, val_s, wrow, wzero, wstage, idxv, wsp):
        c = lax.axis_index("c")
        s = lax.axis_index("s")
        wid = s * 2 + c
        pltpu.sync_copy(idx_hbm, idxv)

        def zrow(r, carry):
            wzero[r, pl.ds(0, 16)] = jnp.zeros((16,), jnp.float32)
            return carry

        lax.fori_loop(0, 128, zrow, 0)

        def run_block(start, nchunks, ntile, so, wbase):
            for t in range(5):
                pltpu.sync_copy(wzero, wsp.at[pl.ds(s * 640 + t * 128, 128)])
            plsc.subcore_barrier()

            def chunk(kk, carry):
                gk = kk * 32 + wid

                @pl.when(gk < nchunks)
                def _():
                    off = start + gk * _C
                    for q in range(4):
                        pltpu.sync_copy(
                            src_hbm.at[pl.ds(off + q * 128, 128)], srcw_s[q])
                    pltpu.sync_copy(dst_hbm.at[pl.ds(off, _C)], dst_s)
                    pltpu.sync_copy(val_hbm.at[pl.ds(off, _C)], val_s)
                    for q in range(4):
                        for j in range(8):
                            sl = srcw_s[q][pl.ds(j * 16, 16)]
                            srcw_s[q][pl.ds(j * 16, 16)] = sl - (so * _N)

                    def group(g, carry2):
                        dst16 = dst_s[pl.ds(g * 16, 16)]
                        val16 = val_s[pl.ds(g * 16, 16)]
                        vm = jnp.where(dst16 == idxv[...], val16, 0.0)
                        for L in range(16):
                            wrow[g * 16 + L, pl.ds(0, 16)] = jnp.broadcast_to(
                                lax.slice(vm, (L,), (L + 1,)), (16,))
                        return carry2

                    lax.fori_loop(0, _C // 16, group, 0)
                    for q in range(4):
                        pltpu.sync_copy(wrow.at[pl.ds(q * 128, 128)],
                                        wsp.at[srcw_s[q]], add=True)

                return carry

            lax.fori_loop(0, ntile, chunk, 0)
            plsc.subcore_barrier()
            pltpu.sync_copy(wsp.at[pl.ds(s * 640, 640)], wstage)
            pltpu.sync_copy(wstage, w_hbm.at[pl.ds(wbase + s * 640, 640)])
            plsc.subcore_barrier()

        def lap_blk(b, carry):
            run_block(b * _E_LAP, _E_LAP // _C, _E_LAP // (32 * _C), b,
                      (c * 10 + b) * _NSC)
            return carry

        def bnd_blk(i, carry):
            so = jnp.where(i < 3, i, i - 2)
            nch = _E_BND // _C
            run_block(4 * _E_LAP + i * _E_BND, nch, (nch + 31) // 32, so,
                      (c * 10 + 4 + i) * _NSC)
            return carry

        lax.fori_loop(0, 4, lap_blk, 0)
        lax.fori_loop(0, 6, bnd_blk, 0)

    return wvec_kernel


def _mm_tanh_body(a_ref, w_ref, o_ref):
    o_ref[:] = jnp.tanh(
        lax.dot_general(a_ref[:], w_ref[:], (((1,), (0,)), ((), ())),
                        preferred_element_type=jnp.float32))


def _mm_tanh(a, w, bm=400):
    m, k = a.shape
    n = w.shape[1]
    return pl.pallas_call(
        _mm_tanh_body,
        grid=(m // bm,),
        in_specs=[
            pl.BlockSpec((bm, k), lambda i: (i, 0)),
            pl.BlockSpec((k, n), lambda i: (0, 0)),
        ],
        out_specs=pl.BlockSpec((bm, n), lambda i: (i, 0)),
        out_shape=jax.ShapeDtypeStruct((m, n), jnp.float32),
    )(a, w)


_NCHUNK = 5
_CKN = _NSC // _NCHUNK


def _t_body(w_ref, e2_ref, o_ref, t_acc):
    b = pl.program_id(0)
    nc = pl.program_id(1)
    wsum = (w_ref[0, 0, :, 0] + w_ref[1, 0, :, 0]).reshape(1, _CKN)
    part = lax.dot_general(wsum, e2_ref[0], (((1,), (0,)), ((), ())),
                           preferred_element_type=jnp.float32)

    @pl.when(nc == 0)
    def _():
        t_acc[b, :] = jnp.zeros((512,), jnp.float32)

    t_acc[b, :] += part[0]

    @pl.when(jnp.logical_and(b == 9, nc == _NCHUNK - 1))
    def _():
        o_ref[:] = t_acc[:]


def _t_kernel(wvec, e2):
    def _so_of(b):
        return jnp.where(b < 4, b, jnp.where(b < 7, b - 4, b - 6))

    return pl.pallas_call(
        _t_body,
        grid=(10, _NCHUNK),
        in_specs=[
            pl.BlockSpec((2, 1, _CKN, 16), lambda b, nc: (0, b, nc, 0)),
            pl.BlockSpec((1, _CKN, 512), lambda b, nc: (_so_of(b), nc, 0)),
        ],
        out_specs=pl.BlockSpec((10, 512), lambda b, nc: (0, 0)),
        out_shape=jax.ShapeDtypeStruct((10, 512), jnp.float32),
        scratch_shapes=[pltpu.VMEM((10, 512), jnp.float32)],
    )(wvec, e2)


def _readout_body(t_ref, m_ref, w3s_ref, w3d_ref, w3u_ref, l1w_ref, l1b_ref,
                  relw_ref, relb_ref, rele_ref, o_ref):
    tm = t_ref[:] * m_ref[:].reshape(10, 1)
    xs = jnp.sum(tm[0:4], axis=0, keepdims=True)
    xd = jnp.sum(tm[4:7], axis=0, keepdims=True)
    xu = jnp.sum(tm[7:10], axis=0, keepdims=True)
    mm = lambda a, w: lax.dot_general(a, w, (((1,), (0,)), ((), ())),
                                      preferred_element_type=jnp.float32)
    h = mm(xs, w3s_ref[:]) + mm(xd, w3d_ref[:]) + mm(xu, w3u_ref[:])
    e3row = jnp.tanh(h)
    final = jnp.tanh(mm(e3row, l1w_ref[:]) + l1b_ref[:])
    s0 = mm(final, relw_ref[0:256])
    srel = mm(rele_ref[:], relw_ref[256:512])
    o_ref[:] = s0[0, 0] + srel + relb_ref[0, 0]


def _readout(t, masks, w3s, w3d, w3u, l1w, l1b, relw, relb, rele):
    full = lambda shape: pl.BlockSpec(shape, lambda: tuple(0 for _ in shape))
    args = (t, masks, w3s, w3d, w3u, l1w, l1b.reshape(1, 256),
            relw, relb.reshape(1, 1), rele)
    return pl.pallas_call(
        _readout_body,
        in_specs=[full(a.shape) for a in args],
        out_specs=full((128, 1)),
        out_shape=jax.ShapeDtypeStruct((128, 1), jnp.float32),
    )(*args)


def _pad_to(x, n, pad_value=0):
    return jnp.concatenate(
        [x, jnp.full((n - x.shape[0],), pad_value, x.dtype)])


def _split_cols(e, nq):
    m = 4 * _N
    return e.reshape(m, nq, _WH).transpose(1, 0, 2).reshape(nq * m, _WH)


def _assemble(z, nq):
    z4 = z.reshape(nq, 10, _NSC, _WH)[:, :, :_N]
    w = nq * _WH
    blk = {}
    for b, (oi, slot, _g, _so) in enumerate(_BLOCKS):
        blk[(oi, slot)] = jnp.concatenate([z4[q, b] for q in range(nq)],
                                          axis=-1)
    zero = jnp.zeros((_N, w), jnp.float32)
    rows = [jnp.concatenate([blk.get((i, sl), zero) for sl in range(3)],
                            axis=-1) for i in range(4)]
    return jnp.stack(rows).reshape(4 * _N, 3 * w)


def kernel(emb0, emb1, emb2, emb3, lap0_idx, lap0_val, lap1_idx, lap1_val,
           lap2_idx, lap2_val, lap3_idx, lap3_val, bnd1_idx, bnd1_val,
           bnd2_idx, bnd2_val, bnd3_idx, bnd3_val, W1s, W1d, W1u, W2s, W2d,
           W2u, W3s, W3d, W3u, lin1_W, lin1_b, rel_W, rel_b, rel_embed,
           order, idx, rel):
    graphs = {}
    for i, (gi, gv) in enumerate([(lap0_idx, lap0_val), (lap1_idx, lap1_val),
                                  (lap2_idx, lap2_val), (lap3_idx, lap3_val)]):
        graphs[('lap', i)] = (gi[0], gi[1], gv)
    for i, (gi, gv) in enumerate([(bnd1_idx, bnd1_val), (bnd2_idx, bnd2_val),
                                  (bnd3_idx, bnd3_val)], start=1):
        graphs[('bnd', i)] = (gi[0], gi[1], gv)
        graphs[('bndT', i)] = (gi[1], gi[0], gv)

    srcs, dsts, vals = [], [], []
    for (oi, slot, gkey, so) in _BLOCKS:
        dst, src, val = graphs[gkey]
        epad = _E_LAP if gkey[0] == 'lap' else _E_BND
        srcs.append(_pad_to(src + so * _N, epad))
        dsts.append(_pad_to(dst, epad))
        vals.append(_pad_to(val, epad, 0.0))
    srcp = jnp.concatenate(srcs)
    dstp = jnp.concatenate(dsts)
    valp = jnp.concatenate(vals)
    idxvec = jnp.full((16,), idx, jnp.int32)

    spmm1 = _make_sc_spmm(1)
    spmm2 = _make_sc_spmm(2)
    wveck = _make_sc_wvec()

    e0 = jnp.stack([emb0, emb1, emb2, emb3])
    z1 = spmm1(_split_cols(e0, 2), srcp, dstp, valp)
    a1 = _assemble(z1, 2)
    e1 = _mm_tanh(a1, jnp.concatenate([W1s, W1d, W1u], axis=0))
    z2 = spmm2(_split_cols(e1.reshape(4, _N, 256), 4), srcp, dstp, valp)
    a2 = _assemble(z2, 4)
    e2 = _mm_tanh(a2, jnp.concatenate([W2s, W2d, W2u], axis=0))

    e2r = e2.reshape(4, _N, 512)
    tv = []
    for (oi, slot, gkey, so) in _BLOCKS:
        dst, src, val = graphs[gkey]
        wv = jnp.zeros((_N,), jnp.float32).at[src].add(
            jnp.where(dst == idx, val, 0.0))
        tv.append(wv @ e2r[so])
    t = jnp.stack(tv)
    oi_arr = jnp.asarray([b[0] for b in _BLOCKS], jnp.int32)
    masks = (oi_arr == order).astype(jnp.float32).reshape(10, 1)
    scores = _readout(t, masks, W3s, W3d, W3u, lin1_W, lin1_b, rel_W,
                      rel_b, rel_embed)
    nz = jnp.nonzero(rel, size=rel.shape[0])[0]
    return scores[nz, 0][:, None]


# speedup vs baseline: 1.0706x; 1.0706x over previous
import functools

import jax
import jax.numpy as jnp
from jax import lax
from jax.experimental import pallas as pl
from jax.experimental.pallas import tpu as pltpu
from jax.experimental.pallas import tpu_sc as plsc

_N = 10000
_NSC = 10240
_E_LAP = 163840
_E_BND = 40960
_E_PAD = 4 * _E_LAP + 6 * _E_BND
_C = 512

_BLOCKS = (
    (0, 0, ('lap', 0), 0),
    (1, 0, ('lap', 1), 1),
    (2, 0, ('lap', 2), 2),
    (3, 0, ('lap', 3), 3),
    (1, 1, ('bndT', 1), 0),
    (2, 1, ('bndT', 2), 1),
    (3, 1, ('bndT', 3), 2),
    (0, 2, ('bnd', 1), 1),
    (1, 2, ('bnd', 2), 2),
    (2, 2, ('bnd', 3), 3),
)
_STARTS = []
_off = 0
for _b in _BLOCKS:
    _STARTS.append(_off)
    _off += _E_LAP if _b[2][0] == 'lap' else _E_BND
_SO = tuple(b[3] for b in _BLOCKS)


_WH = 64


def _make_sc_spmm(npass):
    mesh = plsc.VectorSubcoreMesh(core_axis_name="c", subcore_axis_name="s")
    wh = _WH

    @functools.partial(
        pl.kernel, mesh=mesh,
        out_type=jax.ShapeDtypeStruct((2 * npass * 10 * _NSC, wh),
                                      jnp.float32),
        scratch_types=[
            [pltpu.VMEM((128,), jnp.int32)] * 4,
            [pltpu.VMEM((128,), jnp.int32)] * 4,
            pltpu.VMEM((_C,), jnp.float32),
            pltpu.VMEM((_C, wh), jnp.float32),
            pltpu.VMEM((128, wh), jnp.float32),
            pltpu.VMEM_SHARED((_NSC, wh), jnp.float32),
            pltpu.SemaphoreType.DMA,
        ],
        compiler_params=pltpu.CompilerParams(use_tc_tiling_on_sc=False,
                                             has_side_effects=True),
    )
    def spmm_kernel(x_hbm, src_hbm, dst_hbm, val_hbm, z_hbm,
                    src_s, dst_s, val_s, rows, stage, acc, sem):
        c = lax.axis_index("c")
        s = lax.axis_index("s")

        def zero_stage(_r, carry):
            for j in range(wh // 16):
                stage[_r, pl.ds(j * 16, 16)] = jnp.zeros((16,), jnp.float32)
            return carry

        def scale_group(g, carry):
            val16 = val_s[pl.ds(g * 16, 16)]
            for L in range(16):
                vv = jnp.broadcast_to(lax.slice(val16, (L,), (L + 1,)), (16,))
                r = g * 16 + L
                for j in range(wh // 16):
                    rows[r, pl.ds(j * 16, 16)] = (
                        rows[r, pl.ds(j * 16, 16)] * vv)
            return carry

        def run_block(start, ntile, cofs, zbase):
            lax.fori_loop(0, 128, zero_stage, 0)
            for t in range(5):
                pltpu.sync_copy(stage, acc.at[pl.ds(s * 640 + t * 128, 128)])
            plsc.subcore_barrier()

            def chunk(kk, carry):
                off = start + (kk * 16 + s) * _C
                for q in range(4):
                    pltpu.sync_copy(src_hbm.at[pl.ds(off + q * 128, 128)],
                                    src_s[q])
                    pltpu.sync_copy(dst_hbm.at[pl.ds(off + q * 128, 128)],
                                    dst_s[q])
                pltpu.sync_copy(val_hbm.at[pl.ds(off, _C)], val_s)
                for q in range(4):
                    for j in range(8):
                        sl = src_s[q][pl.ds(j * 16, 16)]
                        src_s[q][pl.ds(j * 16, 16)] = sl + cofs
                for q in range(4):
                    pltpu.async_copy(x_hbm.at[src_s[q]],
                                     rows.at[pl.ds(q * 128, 128)], sem).wait()
                lax.fori_loop(0, _C // 16, scale_group, 0)
                for q in range(4):
                    pltpu.sync_copy(rows.at[pl.ds(q * 128, 128)],
                                    acc.at[dst_s[q]], add=True)
                return carry

            lax.fori_loop(0, ntile, chunk, 0)
            plsc.subcore_barrier()
            for t in range(5):
                r0 = s * 640 + t * 128
                pltpu.sync_copy(acc.at[pl.ds(r0, 128)], stage)
                pltpu.sync_copy(stage, z_hbm.at[pl.ds(zbase + r0, 128)])
            plsc.subcore_barrier()

        for h in range(npass):
            q = c * npass + h
            cofs = q * (4 * _N)

            def lap_blk(b, carry):
                run_block(b * _E_LAP, _E_LAP // (16 * _C), cofs,
                          (q * 10 + b) * _NSC)
                return carry

            def bnd_blk(i, carry):
                run_block(4 * _E_LAP + i * _E_BND, _E_BND // (16 * _C), cofs,
                          (q * 10 + 4 + i) * _NSC)
                return carry

            lax.fori_loop(0, 4, lap_blk, 0)
            lax.fori_loop(0, 6, bnd_blk, 0)

    return spmm_kernel


def _make_sc_wvec():
    mesh = plsc.VectorSubcoreMesh(core_axis_name="c", subcore_axis_name="s")
    wh = _WH

    @functools.partial(
        pl.kernel, mesh=mesh,
        out_type=jax.ShapeDtypeStruct((2 * 10 * _NSC, wh), jnp.float32),
        scratch_types=[
            [pltpu.VMEM((128,), jnp.int32)] * 4,
            pltpu.VMEM((_C,), jnp.int32),
            pltpu.VMEM((_C,), jnp.float32),
            pltpu.VMEM((_C, wh), jnp.float32),
            pltpu.VMEM((128, wh), jnp.float32),
            pltpu.VMEM((128,), jnp.int32),
            pltpu.VMEM_SHARED((_NSC, wh), jnp.float32),
        ],
        compiler_params=pltpu.CompilerParams(use_tc_tiling_on_sc=False,
                                             has_side_effects=True),
    )
    def wvec_kernel(srcraw_hbm, dst_hbm, val_hbm, idx_hbm, w_hbm,
                    srcw_s, dst_s, val_s, wrow, wstage, idxv, wsp):
        c = lax.axis_index("c")
        s = lax.axis_index("s")
        wid = s * 2 + c
        pltpu.sync_copy(idx_hbm, idxv)

        def zrow(r, carry):
            for j in range(wh // 16):
                wstage[r, pl.ds(j * 16, 16)] = jnp.zeros((16,), jnp.float32)
            return carry

        def run_block(start, nchunks, ntile, wbase):
            lax.fori_loop(0, 128, zrow, 0)
            for t in range(5):
                pltpu.sync_copy(wstage, wsp.at[pl.ds(s * 640 + t * 128, 128)])
            plsc.subcore_barrier()

            def chunk(kk, carry):
                gk = kk * 32 + wid
                ok = (gk < nchunks).astype(jnp.float32)
                gkc = jnp.minimum(gk, nchunks - 1)
                off = start + gkc * _C
                for q in range(4):
                    pltpu.sync_copy(
                        srcraw_hbm.at[pl.ds(off + q * 128, 128)], srcw_s[q])
                pltpu.sync_copy(dst_hbm.at[pl.ds(off, _C)], dst_s)
                pltpu.sync_copy(val_hbm.at[pl.ds(off, _C)], val_s)

                idx16 = idxv[pl.ds(0, 16)]
                okv = jnp.broadcast_to(ok, (16,))

                def group(g, carry2):
                    dst16 = dst_s[pl.ds(g * 16, 16)]
                    val16 = val_s[pl.ds(g * 16, 16)]
                    vm = jnp.where(dst16 == idx16, val16, 0.0) * okv
                    for L in range(16):
                        bc = jnp.broadcast_to(
                            lax.slice(vm, (L,), (L + 1,)), (16,))
                        for j in range(wh // 16):
                            wrow[g * 16 + L, pl.ds(j * 16, 16)] = bc
                    return carry2

                lax.fori_loop(0, _C // 16, group, 0)
                for q in range(4):
                    pltpu.sync_copy(wrow.at[pl.ds(q * 128, 128)],
                                    wsp.at[srcw_s[q]], add=True)
                return carry

            lax.fori_loop(0, ntile, chunk, 0)
            plsc.subcore_barrier()
            for t in range(5):
                r0 = s * 640 + t * 128
                pltpu.sync_copy(wsp.at[pl.ds(r0, 128)], wstage)
                pltpu.sync_copy(wstage, w_hbm.at[pl.ds(wbase + r0, 128)])
            plsc.subcore_barrier()

        def lap_blk(b, carry):
            run_block(b * _E_LAP, _E_LAP // _C, _E_LAP // (32 * _C),
                      (c * 10 + b) * _NSC)
            return carry

        def bnd_blk(i, carry):
            nch = _E_BND // _C
            run_block(4 * _E_LAP + i * _E_BND, nch, (nch + 31) // 32,
                      (c * 10 + 4 + i) * _NSC)
            return carry

        lax.fori_loop(0, 4, lap_blk, 0)
        lax.fori_loop(0, 6, bnd_blk, 0)

    return wvec_kernel


def _mm_tanh_body(a_ref, w_ref, o_ref):
    o_ref[:] = jnp.tanh(
        lax.dot_general(a_ref[:], w_ref[:], (((1,), (0,)), ((), ())),
                        preferred_element_type=jnp.float32))


def _mm_tanh(a, w, bm=400):
    m, k = a.shape
    n = w.shape[1]
    return pl.pallas_call(
        _mm_tanh_body,
        grid=(m // bm,),
        in_specs=[
            pl.BlockSpec((bm, k), lambda i: (i, 0)),
            pl.BlockSpec((k, n), lambda i: (0, 0)),
        ],
        out_specs=pl.BlockSpec((bm, n), lambda i: (i, 0)),
        out_shape=jax.ShapeDtypeStruct((m, n), jnp.float32),
    )(a, w)


_NCHUNK = 5
_CKN = _NSC // _NCHUNK


def _t_body(w_ref, e2_ref, o_ref, t_acc):
    b = pl.program_id(0)
    nc = pl.program_id(1)
    wsum = (w_ref[0, 0, :, 0] + w_ref[1, 0, :, 0]).reshape(1, _CKN)
    part = lax.dot_general(wsum, e2_ref[0], (((1,), (0,)), ((), ())),
                           preferred_element_type=jnp.float32)

    @pl.when(nc == 0)
    def _():
        t_acc[b, :] = jnp.zeros((512,), jnp.float32)

    t_acc[b, :] += part[0]

    @pl.when(jnp.logical_and(b == 9, nc == _NCHUNK - 1))
    def _():
        o_ref[:] = t_acc[:]


def _t_kernel(wvec, e2):
    def _so_of(b):
        return jnp.where(b < 4, b, jnp.where(b < 7, b - 4, b - 6))

    return pl.pallas_call(
        _t_body,
        grid=(10, _NCHUNK),
        in_specs=[
            pl.BlockSpec((2, 1, _CKN, _WH), lambda b, nc: (0, b, nc, 0)),
            pl.BlockSpec((1, _CKN, 512), lambda b, nc: (_so_of(b), nc, 0)),
        ],
        out_specs=pl.BlockSpec((10, 512), lambda b, nc: (0, 0)),
        out_shape=jax.ShapeDtypeStruct((10, 512), jnp.float32),
        scratch_shapes=[pltpu.VMEM((10, 512), jnp.float32)],
    )(wvec, e2)


def _readout_body(t_ref, m_ref, w3s_ref, w3d_ref, w3u_ref, l1w_ref, l1b_ref,
                  relw_ref, relb_ref, rele_ref, o_ref):
    tm = t_ref[:] * m_ref[:].reshape(10, 1)
    xs = jnp.sum(tm[0:4], axis=0, keepdims=True)
    xd = jnp.sum(tm[4:7], axis=0, keepdims=True)
    xu = jnp.sum(tm[7:10], axis=0, keepdims=True)
    mm = lambda a, w: lax.dot_general(a, w, (((1,), (0,)), ((), ())),
                                      preferred_element_type=jnp.float32)
    h = mm(xs, w3s_ref[:]) + mm(xd, w3d_ref[:]) + mm(xu, w3u_ref[:])
    e3row = jnp.tanh(h)
    final = jnp.tanh(mm(e3row, l1w_ref[:]) + l1b_ref[:])
    s0 = mm(final, relw_ref[0:256])
    srel = mm(rele_ref[:], relw_ref[256:512])
    o_ref[:] = s0[0, 0] + srel + relb_ref[0, 0]


def _readout(t, masks, w3s, w3d, w3u, l1w, l1b, relw, relb, rele):
    full = lambda shape: pl.BlockSpec(shape, lambda: tuple(0 for _ in shape))
    args = (t, masks, w3s, w3d, w3u, l1w, l1b.reshape(1, 256),
            relw, relb.reshape(1, 1), rele)
    return pl.pallas_call(
        _readout_body,
        in_specs=[full(a.shape) for a in args],
        out_specs=full((128, 1)),
        out_shape=jax.ShapeDtypeStruct((128, 1), jnp.float32),
    )(*args)


def _pad_to(x, n, pad_value=0):
    return jnp.concatenate(
        [x, jnp.full((n - x.shape[0],), pad_value, x.dtype)])


def _split_cols(e, nq):
    m = 4 * _N
    return e.reshape(m, nq, _WH).transpose(1, 0, 2).reshape(nq * m, _WH)


def _assemble(z, nq):
    z4 = z.reshape(nq, 10, _NSC, _WH)[:, :, :_N]
    w = nq * _WH
    blk = {}
    for b, (oi, slot, _g, _so) in enumerate(_BLOCKS):
        blk[(oi, slot)] = jnp.concatenate([z4[q, b] for q in range(nq)],
                                          axis=-1)
    zero = jnp.zeros((_N, w), jnp.float32)
    rows = [jnp.concatenate([blk.get((i, sl), zero) for sl in range(3)],
                            axis=-1) for i in range(4)]
    return jnp.stack(rows).reshape(4 * _N, 3 * w)


def kernel(emb0, emb1, emb2, emb3, lap0_idx, lap0_val, lap1_idx, lap1_val,
           lap2_idx, lap2_val, lap3_idx, lap3_val, bnd1_idx, bnd1_val,
           bnd2_idx, bnd2_val, bnd3_idx, bnd3_val, W1s, W1d, W1u, W2s, W2d,
           W2u, W3s, W3d, W3u, lin1_W, lin1_b, rel_W, rel_b, rel_embed,
           order, idx, rel):
    graphs = {}
    for i, (gi, gv) in enumerate([(lap0_idx, lap0_val), (lap1_idx, lap1_val),
                                  (lap2_idx, lap2_val), (lap3_idx, lap3_val)]):
        graphs[('lap', i)] = (gi[0], gi[1], gv)
    for i, (gi, gv) in enumerate([(bnd1_idx, bnd1_val), (bnd2_idx, bnd2_val),
                                  (bnd3_idx, bnd3_val)], start=1):
        graphs[('bnd', i)] = (gi[0], gi[1], gv)
        graphs[('bndT', i)] = (gi[1], gi[0], gv)

    srcs, dsts, vals, srcr = [], [], [], []
    for (oi, slot, gkey, so) in _BLOCKS:
        dst, src, val = graphs[gkey]
        epad = _E_LAP if gkey[0] == 'lap' else _E_BND
        srcs.append(_pad_to(src + so * _N, epad))
        srcr.append(_pad_to(src, epad))
        dsts.append(_pad_to(dst, epad))
        vals.append(_pad_to(val, epad, 0.0))
    srcp = jnp.concatenate(srcs)
    srcraw = jnp.concatenate(srcr)
    dstp = jnp.concatenate(dsts)
    valp = jnp.concatenate(vals)
    idxvec = jnp.full((128,), idx, jnp.int32)

    spmm1 = _make_sc_spmm(1)
    spmm2 = _make_sc_spmm(2)
    wveck = _make_sc_wvec()

    e0 = jnp.stack([emb0, emb1, emb2, emb3])
    z1 = spmm1(_split_cols(e0, 2), srcp, dstp, valp)
    a1 = _assemble(z1, 2)
    e1 = _mm_tanh(a1, jnp.concatenate([W1s, W1d, W1u], axis=0))
    z2 = spmm2(_split_cols(e1.reshape(4, _N, 256), 4), srcp, dstp, valp)
    a2 = _assemble(z2, 4)
    e2 = _mm_tanh(a2, jnp.concatenate([W2s, W2d, W2u], axis=0))

    idxvec = idxvec + (z2.reshape(-1)[0] * 0.0).astype(jnp.int32)
    wvec = wveck(srcraw, dstp, valp, idxvec).reshape(2, 10, _NSC, _WH)
    e2p = jnp.pad(e2.reshape(4, _N, 512), ((0, 0), (0, _NSC - _N), (0, 0)))
    t = _t_kernel(wvec, e2p)
    oi_arr = jnp.asarray([b[0] for b in _BLOCKS], jnp.int32)
    masks = (oi_arr == order).astype(jnp.float32).reshape(10, 1)
    scores = _readout(t, masks, W3s, W3d, W3u, lin1_W, lin1_b, rel_W,
                      rel_b, rel_embed)
    nz = jnp.nonzero(rel, size=rel.shape[0])[0]
    return scores[nz, 0][:, None]
